# baseline (speedup 1.0000x reference)
import functools

import jax
import jax.numpy as jnp
from jax import lax
from jax.experimental import pallas as pl
from jax.experimental.pallas import tpu as pltpu
from jax.experimental.pallas import tpu_sc as plsc

_POOLED_H = 7
_POOLED_W = 7
_POOLED_L = 4
_TEMPORAL_SCALE = 0.125

_NT = 288


def _floor_i32(x):
    t = x.astype(jnp.int32)
    return jnp.where(t.astype(jnp.float32) > x, t - 1, t)


def _round_f32(x):
    big = jnp.float32(12582912.0)
    r = (x + big) - big
    return jnp.where(jnp.abs(x) >= jnp.float32(4194304.0), x, r)


def _bins_math(start_f, end_f, num_l, use_sc_round):
    rnd = _round_f32 if use_sc_round else jnp.round
    flr = _floor_i32 if use_sc_round else (
        lambda x: jnp.floor(x).astype(jnp.int32)
    )
    start_l = rnd(start_f * _TEMPORAL_SCALE).astype(jnp.int32)
    end_l = rnd(end_f * _TEMPORAL_SCALE).astype(jnp.int32)
    roi_length = jnp.maximum(end_l - start_l + 1, 1)
    bin_size_l = roi_length.astype(jnp.float32) * (1.0 / _POOLED_L)
    vals = []
    for p in range(_POOLED_L):
        ls = jnp.clip(flr(p * bin_size_l) + start_l, 0, num_l)
        le = jnp.clip(flr((p + 1) * bin_size_l) + start_l, 0, num_l)
        vals.append(
            jnp.where(ls <= le, jnp.float32(0.0), jnp.float32(-jnp.inf))
        )
    return vals


def _tc_bins_kernel(rois_ref, out_ref, *, num_l):
    rois = rois_ref[0]
    vals = _bins_math(rois[:, 5:6], rois[:, 6:7], num_l, use_sc_round=False)
    out_ref[...] = jnp.concatenate(vals, axis=1)


def _sc_bins_body(num_l, nc, start_hbm, end_hbm, out_hbm, sv, ev, outv):
    w = lax.axis_index("s") * nc + lax.axis_index("c")
    base = w * 16
    pltpu.sync_copy(start_hbm.at[pl.ds(base, 16)], sv)
    pltpu.sync_copy(end_hbm.at[pl.ds(base, 16)], ev)
    vals = _bins_math(sv[...], ev[...], num_l, use_sc_round=True)
    cap = out_hbm.shape[0] // _POOLED_L
    for p in range(_POOLED_L):
        outv[...] = vals[p]
        pltpu.sync_copy(outv, out_hbm.at[pl.ds(p * cap + base, 16)])


def kernel(features, rois):
    B, C, L, H, W = features.shape
    num_rois = rois.shape[0]
    nt = min(_NT, num_rois)
    n_sc = num_rois - nt

    rois_head = rois[:nt].reshape(1, nt, 7)
    bins_tc = pl.pallas_call(
        functools.partial(_tc_bins_kernel, num_l=L),
        in_specs=[pl.BlockSpec((1, nt, 7), lambda: (0, 0, 0))],
        out_specs=pl.BlockSpec((nt, _POOLED_L), lambda: (0, 0)),
        out_shape=jax.ShapeDtypeStruct((nt, _POOLED_L), jnp.float32),
    )(rois_head)

    out_shape = (num_rois, C, _POOLED_L, _POOLED_H, _POOLED_W)
    head = jnp.broadcast_to(
        bins_tc[:, None, :, None, None], (nt,) + out_shape[1:]
    )
    if n_sc == 0:
        return head

    info = plsc.get_sparse_core_info()
    nc, ns = info.num_cores, info.num_subcores
    lanes = 16
    cap = nc * ns * lanes
    assert cap >= n_sc

    start_col = jnp.pad(rois[nt:, 5], (0, cap - n_sc))
    end_col = jnp.pad(rois[nt:, 6], (0, cap - n_sc))

    mesh = plsc.VectorSubcoreMesh(core_axis_name="c", subcore_axis_name="s")
    bins_sc_flat = pl.kernel(
        functools.partial(_sc_bins_body, L, nc),
        mesh=mesh,
        out_type=jax.ShapeDtypeStruct((_POOLED_L * cap,), jnp.float32),
        scratch_types=[
            pltpu.VMEM((lanes,), jnp.float32),
            pltpu.VMEM((lanes,), jnp.float32),
            pltpu.VMEM((lanes,), jnp.float32),
        ],
    )(start_col, end_col)

    bins_sc = bins_sc_flat.reshape(_POOLED_L, cap)[:, :n_sc].T
    tail = jnp.broadcast_to(
        bins_sc[:, None, :, None, None], (n_sc,) + out_shape[1:]
    )
    return jnp.concatenate([head, tail], axis=0)
